# Initial kernel scaffold; baseline (speedup 1.0000x reference)
import jax
import jax.numpy as jnp
from jax.experimental import pallas as pl


def kernel(x, adj_lists, W1, b1, W2, b2):
    raise NotImplementedError("write your pallas kernel here")


# speedup vs baseline: 1.1279x; 1.1279x over previous
import functools

import jax
import jax.numpy as jnp
from jax import lax
from jax.experimental import pallas as pl
from jax.experimental.pallas import tpu as pltpu
from jax.experimental.pallas import tpu_sc as plsc

NN = 10000
DG = 16
DD = 256
NW = 32
NPAD = 10240
PER_W = NPAD // NW
CHUNK = 8
NCHUNK = PER_W // CHUNK
LANES = 16


def _sc_gather_mean(table, idx):
    mesh = plsc.VectorSubcoreMesh(core_axis_name="c", subcore_axis_name="s")

    @functools.partial(
        pl.kernel,
        mesh=mesh,
        out_type=jax.ShapeDtypeStruct((NPAD, DD), jnp.float32),
        scratch_types=[
            pltpu.VMEM((PER_W * DG,), jnp.int32),
            pltpu.VMEM((CHUNK * DG, DD), jnp.float32),
            pltpu.VMEM((CHUNK, DD), jnp.float32),
            pltpu.SemaphoreType.DMA,
        ],
    )
    def k(table_hbm, idx_hbm, out_hbm, idx_v, rows_v, acc_v, sem):
        wid = lax.axis_index("s") * 2 + lax.axis_index("c")
        base = wid * PER_W
        pltpu.sync_copy(idx_hbm.at[pl.ds(base * DG, PER_W * DG)], idx_v)

        def chunk_body(c, _):
            cp = pltpu.async_copy(
                table_hbm.at[idx_v.at[pl.ds(c * CHUNK * DG, CHUNK * DG)]],
                rows_v, sem)
            cp.wait()

            def node_body(n, _):
                r0 = n * DG
                for d in range(DD // LANES):
                    sl = pl.ds(d * LANES, LANES)
                    acc = rows_v[r0, sl]
                    for j in range(1, DG):
                        acc = acc + rows_v[r0 + j, sl]
                    acc_v[n, sl] = acc * (1.0 / DG)
                return 0

            lax.fori_loop(0, CHUNK, node_body, 0)
            pltpu.sync_copy(acc_v, out_hbm.at[pl.ds(base + c * CHUNK, CHUNK)])
            return 0

        lax.fori_loop(0, NCHUNK, chunk_body, 0)

    return k(table, idx)


def _tc_linear(h, agg, w_self, w_neigh, b):
    blk = 512

    def body(h_ref, a_ref, ws_ref, wn_ref, b_ref, o_ref):
        acc = jnp.dot(h_ref[...], ws_ref[...],
                      preferred_element_type=jnp.float32)
        acc = acc + jnp.dot(a_ref[...], wn_ref[...],
                            preferred_element_type=jnp.float32)
        o_ref[...] = jnp.maximum(acc + b_ref[...], 0.0)

    return pl.pallas_call(
        body,
        grid=(NPAD // blk,),
        in_specs=[
            pl.BlockSpec((blk, DD), lambda i: (i, 0)),
            pl.BlockSpec((blk, DD), lambda i: (i, 0)),
            pl.BlockSpec((DD, DD), lambda i: (0, 0)),
            pl.BlockSpec((DD, DD), lambda i: (0, 0)),
            pl.BlockSpec((1, DD), lambda i: (0, 0)),
        ],
        out_specs=pl.BlockSpec((blk, DD), lambda i: (i, 0)),
        out_shape=jax.ShapeDtypeStruct((NPAD, DD), jnp.float32),
    )(h, agg, w_self, w_neigh, b)


def kernel(x, adj_lists, W1, b1, W2, b2):
    idx = adj_lists.astype(jnp.int32).reshape(-1)
    idx = jnp.pad(idx, (0, (NPAD - NN) * DG))
    h = jnp.pad(x, ((0, NPAD - NN), (0, 0)))

    for W, b in ((W1, b1), (W2, b2)):
        wt = W.T
        agg = _sc_gather_mean(h, idx)
        h = _tc_linear(h, agg, wt[:DD], wt[DD:], b.reshape(1, DD))
    return h[:NN]


# speedup vs baseline: 1.5105x; 1.3393x over previous
import functools

import jax
import jax.numpy as jnp
from jax import lax
from jax.experimental import pallas as pl
from jax.experimental.pallas import tpu as pltpu
from jax.experimental.pallas import tpu_sc as plsc

NN = 10000
DG = 16
DD = 256
NW = 32
NPAD = 10240
PER_W = NPAD // NW
CHUNK = 8
NCHUNK = PER_W // CHUNK
LANES = 16


def _sc_gather_sum(table, idx):
    mesh = plsc.VectorSubcoreMesh(core_axis_name="c", subcore_axis_name="s")

    @functools.partial(
        pl.kernel,
        mesh=mesh,
        out_type=jax.ShapeDtypeStruct((NPAD, DD), jnp.float32),
        scratch_types=[
            pltpu.VMEM((PER_W * DG,), jnp.int32),
            pltpu.VMEM((2, CHUNK * DG, DD), jnp.float32),
            pltpu.VMEM((2, CHUNK, DD), jnp.float32),
            pltpu.SemaphoreType.DMA,
            pltpu.SemaphoreType.DMA,
            pltpu.SemaphoreType.DMA,
            pltpu.SemaphoreType.DMA,
        ],
    )
    def k(table_hbm, idx_hbm, out_hbm, idx_v, rows_v, acc_v,
          gsem0, gsem1, osem0, osem1):
        wid = lax.axis_index("s") * 2 + lax.axis_index("c")
        base = wid * PER_W
        pltpu.sync_copy(idx_hbm.at[pl.ds(base * DG, PER_W * DG)], idx_v)

        gsems = (gsem0, gsem1)
        osems = (osem0, osem1)

        def issue_gather(c, par):
            return pltpu.async_copy(
                table_hbm.at[idx_v.at[pl.ds(c * CHUNK * DG, CHUNK * DG)]],
                rows_v.at[par], gsems[par])

        def reduce_slab(par):
            def node_body(n, _):
                r0 = n * DG
                for d in range(DD // LANES):
                    sl = pl.ds(d * LANES, LANES)
                    vals = [rows_v[par, r0 + j, sl] for j in range(DG)]
                    while len(vals) > 1:
                        vals = [vals[i] + vals[i + 1]
                                for i in range(0, len(vals), 2)]
                    acc_v[par, n, sl] = vals[0]
                return 0
            lax.fori_loop(0, CHUNK, node_body, 0)

        issue_gather(0, 0)
        issue_gather(1, 1)

        def wait_gather(par):
            pltpu.make_async_copy(
                table_hbm.at[pl.ds(0, CHUNK * DG)],
                rows_v.at[par], gsems[par]).wait()

        def pair_body(t, _):
            for par in range(2):
                c = 2 * t + par
                wait_gather(par)

                @pl.when(t > 0)
                def _():
                    pltpu.make_async_copy(
                        acc_v.at[par],
                        out_hbm.at[pl.ds(base, CHUNK)], osems[par]).wait()

                reduce_slab(par)

                @pl.when(c + 2 < NCHUNK)
                def _():
                    issue_gather(c + 2, par)

                pltpu.async_copy(
                    acc_v.at[par],
                    out_hbm.at[pl.ds(base + c * CHUNK, CHUNK)], osems[par])
            return 0

        lax.fori_loop(0, NCHUNK // 2, pair_body, 0)
        for par in range(2):
            pltpu.make_async_copy(
                acc_v.at[par], out_hbm.at[pl.ds(base, CHUNK)],
                osems[par]).wait()

    return k(table, idx)


def _tc_linear(h, agg, w_self, w_neigh, b):
    blk = 512

    def body(h_ref, a_ref, ws_ref, wn_ref, b_ref, o_ref):
        acc = jnp.dot(h_ref[...], ws_ref[...],
                      preferred_element_type=jnp.float32)
        acc = acc + jnp.dot(a_ref[...], wn_ref[...],
                            preferred_element_type=jnp.float32)
        o_ref[...] = jnp.maximum(acc + b_ref[...], 0.0)

    return pl.pallas_call(
        body,
        grid=(NPAD // blk,),
        in_specs=[
            pl.BlockSpec((blk, DD), lambda i: (i, 0)),
            pl.BlockSpec((blk, DD), lambda i: (i, 0)),
            pl.BlockSpec((DD, DD), lambda i: (0, 0)),
            pl.BlockSpec((DD, DD), lambda i: (0, 0)),
            pl.BlockSpec((1, DD), lambda i: (0, 0)),
        ],
        out_specs=pl.BlockSpec((blk, DD), lambda i: (i, 0)),
        out_shape=jax.ShapeDtypeStruct((NPAD, DD), jnp.float32),
    )(h, agg, w_self, w_neigh, b)


def kernel(x, adj_lists, W1, b1, W2, b2):
    idx = adj_lists.astype(jnp.int32).reshape(-1)
    idx = jnp.pad(idx, (0, (NPAD - NN) * DG))
    h = jnp.pad(x, ((0, NPAD - NN), (0, 0)))

    for W, b in ((W1, b1), (W2, b2)):
        wt = W.T
        agg = _sc_gather_sum(h, idx)
        h = _tc_linear(h, agg, wt[:DD], wt[DD:] * (1.0 / DG),
                       b.reshape(1, DD))
    return h[:NN]
